# Initial kernel scaffold; baseline (speedup 1.0000x reference)
import jax
import jax.numpy as jnp
from jax.experimental import pallas as pl


def kernel(x, rel_pos_emb):
    raise NotImplementedError("write your pallas kernel here")


# speedup vs baseline: 1.4314x; 1.4314x over previous
import functools

import jax
import jax.numpy as jnp
from jax.experimental import pallas as pl
from jax.experimental.pallas import tpu as pltpu

_MAX_LEN = 2048
_S_BLK = 256


def _body(x_ref, emb_hbm, o_ref, emb_vmem, tail_vmem, sem, sem2, *, seq_len, base, shift):
    i = pl.program_id(0)
    b = pl.program_id(1)

    @pl.when((i == 0) & (b == 0))
    def _fetch_rows():
        cp1 = pltpu.make_async_copy(
            emb_hbm.at[pl.ds(base, seq_len), :],
            emb_vmem.at[pl.ds(0, seq_len), :],
            sem,
        )
        cp1.start()
        cp2 = pltpu.make_async_copy(
            emb_hbm.at[pl.ds(base + seq_len, shift), :], tail_vmem, sem2
        )
        cp2.start()
        cp1.wait()
        cp2.wait()
        pad = jnp.zeros((8 - shift, tail_vmem.shape[1]), tail_vmem.dtype)
        emb_vmem[pl.ds(seq_len, 8), :] = jnp.concatenate(
            [tail_vmem[...], pad], axis=0
        )

    win = emb_vmem[pl.ds(i * _S_BLK, _S_BLK + 8), :]
    rows = jax.lax.slice(win, (shift, 0), (shift + _S_BLK, win.shape[1]))
    o_ref[...] = x_ref[...] + rows[None, :, :]


def kernel(x, rel_pos_emb):
    batch, seq_len, d_model = x.shape
    n_blk = seq_len // _S_BLK
    base = (_MAX_LEN - 1) // 8 * 8
    shift = (_MAX_LEN - 1) - base
    body = functools.partial(_body, seq_len=seq_len, base=base, shift=shift)
    return pl.pallas_call(
        body,
        grid=(n_blk, batch),
        in_specs=[
            pl.BlockSpec((1, _S_BLK, d_model), lambda i, b: (b, i, 0)),
            pl.BlockSpec(memory_space=pltpu.MemorySpace.HBM),
        ],
        out_specs=pl.BlockSpec((1, _S_BLK, d_model), lambda i, b: (b, i, 0)),
        out_shape=jax.ShapeDtypeStruct(x.shape, x.dtype),
        scratch_shapes=[
            pltpu.VMEM((seq_len + 8, d_model), x.dtype),
            pltpu.VMEM((shift, d_model), x.dtype),
            pltpu.SemaphoreType.DMA,
            pltpu.SemaphoreType.DMA,
        ],
    )(x, rel_pos_emb)


# speedup vs baseline: 1.8967x; 1.3251x over previous
import functools

import jax
import jax.numpy as jnp
from jax.experimental import pallas as pl
from jax.experimental.pallas import tpu as pltpu

_MAX_LEN = 2048
_S_BLK = 512


def _body(x_ref, emb_hbm, o_ref, emb_vmem, tail_vmem, sem, sem2, *, seq_len, base, shift):
    i = pl.program_id(0)
    b = pl.program_id(1)

    @pl.when((i == 0) & (b == 0))
    def _fetch_rows():
        cp1 = pltpu.make_async_copy(
            emb_hbm.at[pl.ds(base, seq_len), :],
            emb_vmem.at[pl.ds(0, seq_len), :],
            sem,
        )
        cp1.start()
        cp2 = pltpu.make_async_copy(
            emb_hbm.at[pl.ds(base + seq_len, shift), :], tail_vmem, sem2
        )
        cp2.start()
        cp1.wait()
        cp2.wait()
        pad = jnp.zeros((8 - shift, tail_vmem.shape[1]), tail_vmem.dtype)
        emb_vmem[pl.ds(seq_len, 8), :] = jnp.concatenate(
            [tail_vmem[...], pad], axis=0
        )

    win = emb_vmem[pl.ds(i * _S_BLK, _S_BLK + 8), :]
    rows = jax.lax.slice(win, (shift, 0), (shift + _S_BLK, win.shape[1]))
    o_ref[...] = x_ref[...] + rows[None, :, :]


def kernel(x, rel_pos_emb):
    batch, seq_len, d_model = x.shape
    n_blk = seq_len // _S_BLK
    base = (_MAX_LEN - 1) // 8 * 8
    shift = (_MAX_LEN - 1) - base
    body = functools.partial(_body, seq_len=seq_len, base=base, shift=shift)
    return pl.pallas_call(
        body,
        grid=(n_blk, batch),
        in_specs=[
            pl.BlockSpec((1, _S_BLK, d_model), lambda i, b: (b, i, 0)),
            pl.BlockSpec(memory_space=pltpu.MemorySpace.HBM),
        ],
        out_specs=pl.BlockSpec((1, _S_BLK, d_model), lambda i, b: (b, i, 0)),
        out_shape=jax.ShapeDtypeStruct(x.shape, x.dtype),
        scratch_shapes=[
            pltpu.VMEM((seq_len + 8, d_model), x.dtype),
            pltpu.VMEM((shift, d_model), x.dtype),
            pltpu.SemaphoreType.DMA,
            pltpu.SemaphoreType.DMA,
        ],
    )(x, rel_pos_emb)


# speedup vs baseline: 2.0541x; 1.0830x over previous
import functools

import jax
import jax.numpy as jnp
from jax.experimental import pallas as pl
from jax.experimental.pallas import tpu as pltpu

_MAX_LEN = 2048
_S_BLK = 1024


def _body(x_ref, emb_hbm, o_ref, emb_vmem, tail_vmem, sem, sem2, *, seq_len, base, shift):
    i = pl.program_id(0)
    b = pl.program_id(1)

    @pl.when((i == 0) & (b == 0))
    def _fetch_rows():
        cp1 = pltpu.make_async_copy(
            emb_hbm.at[pl.ds(base, seq_len), :],
            emb_vmem.at[pl.ds(0, seq_len), :],
            sem,
        )
        cp1.start()
        cp2 = pltpu.make_async_copy(
            emb_hbm.at[pl.ds(base + seq_len, shift), :], tail_vmem, sem2
        )
        cp2.start()
        cp1.wait()
        cp2.wait()
        pad = jnp.zeros((8 - shift, tail_vmem.shape[1]), tail_vmem.dtype)
        emb_vmem[pl.ds(seq_len, 8), :] = jnp.concatenate(
            [tail_vmem[...], pad], axis=0
        )

    win = emb_vmem[pl.ds(i * _S_BLK, _S_BLK + 8), :]
    rows = jax.lax.slice(win, (shift, 0), (shift + _S_BLK, win.shape[1]))
    o_ref[...] = x_ref[...] + rows[None, :, :]


def kernel(x, rel_pos_emb):
    batch, seq_len, d_model = x.shape
    n_blk = seq_len // _S_BLK
    base = (_MAX_LEN - 1) // 8 * 8
    shift = (_MAX_LEN - 1) - base
    body = functools.partial(_body, seq_len=seq_len, base=base, shift=shift)
    return pl.pallas_call(
        body,
        grid=(n_blk, batch),
        in_specs=[
            pl.BlockSpec((1, _S_BLK, d_model), lambda i, b: (b, i, 0)),
            pl.BlockSpec(memory_space=pltpu.MemorySpace.HBM),
        ],
        out_specs=pl.BlockSpec((1, _S_BLK, d_model), lambda i, b: (b, i, 0)),
        out_shape=jax.ShapeDtypeStruct(x.shape, x.dtype),
        scratch_shapes=[
            pltpu.VMEM((seq_len + 8, d_model), x.dtype),
            pltpu.VMEM((shift, d_model), x.dtype),
            pltpu.SemaphoreType.DMA,
            pltpu.SemaphoreType.DMA,
        ],
    )(x, rel_pos_emb)


# speedup vs baseline: 2.1063x; 1.0254x over previous
import functools

import jax
import jax.numpy as jnp
from jax.experimental import pallas as pl
from jax.experimental.pallas import tpu as pltpu

_MAX_LEN = 2048
_S_BLK = 2048


def _body(x_ref, emb_hbm, o_ref, emb_vmem, tail_vmem, sem, sem2, *, seq_len, base, shift):
    i = pl.program_id(0)
    b = pl.program_id(1)

    @pl.when((i == 0) & (b == 0))
    def _fetch_rows():
        cp1 = pltpu.make_async_copy(
            emb_hbm.at[pl.ds(base, seq_len), :],
            emb_vmem.at[pl.ds(0, seq_len), :],
            sem,
        )
        cp1.start()
        cp2 = pltpu.make_async_copy(
            emb_hbm.at[pl.ds(base + seq_len, shift), :], tail_vmem, sem2
        )
        cp2.start()
        cp1.wait()
        cp2.wait()
        pad = jnp.zeros((8 - shift, tail_vmem.shape[1]), tail_vmem.dtype)
        emb_vmem[pl.ds(seq_len, 8), :] = jnp.concatenate(
            [tail_vmem[...], pad], axis=0
        )

    win = emb_vmem[pl.ds(i * _S_BLK, _S_BLK + 8), :]
    rows = jax.lax.slice(win, (shift, 0), (shift + _S_BLK, win.shape[1]))
    o_ref[...] = x_ref[...] + rows[None, :, :]


def kernel(x, rel_pos_emb):
    batch, seq_len, d_model = x.shape
    n_blk = seq_len // _S_BLK
    base = (_MAX_LEN - 1) // 8 * 8
    shift = (_MAX_LEN - 1) - base
    body = functools.partial(_body, seq_len=seq_len, base=base, shift=shift)
    return pl.pallas_call(
        body,
        grid=(n_blk, batch),
        in_specs=[
            pl.BlockSpec((1, _S_BLK, d_model), lambda i, b: (b, i, 0)),
            pl.BlockSpec(memory_space=pltpu.MemorySpace.HBM),
        ],
        out_specs=pl.BlockSpec((1, _S_BLK, d_model), lambda i, b: (b, i, 0)),
        out_shape=jax.ShapeDtypeStruct(x.shape, x.dtype),
        scratch_shapes=[
            pltpu.VMEM((seq_len + 8, d_model), x.dtype),
            pltpu.VMEM((shift, d_model), x.dtype),
            pltpu.SemaphoreType.DMA,
            pltpu.SemaphoreType.DMA,
        ],
    )(x, rel_pos_emb)


# speedup vs baseline: 2.1335x; 1.0129x over previous
import functools

import jax
import jax.numpy as jnp
from jax.experimental import pallas as pl
from jax.experimental.pallas import tpu as pltpu

_MAX_LEN = 2048


def _half(x_ref, o_ref, emb_vmem, h0, hrows, shift):
    win = emb_vmem[pl.ds(h0, hrows + 8), :]
    rows = jax.lax.slice(win, (shift, 0), (shift + hrows, win.shape[1]))
    o_ref[0, pl.ds(h0, hrows), :] = x_ref[0, pl.ds(h0, hrows), :] + rows


def _body(x_ref, emb_hbm, o_ref, emb_vmem, tail_vmem, sem_a, sem_b, sem_t,
          *, seq_len, base, shift):
    b = pl.program_id(1)
    half = seq_len // 2
    a_rows = half + 8
    b_rows = seq_len - a_rows

    @pl.when(b == 0)
    def _start_dmas():
        pltpu.make_async_copy(
            emb_hbm.at[pl.ds(base, a_rows), :],
            emb_vmem.at[pl.ds(0, a_rows), :],
            sem_a,
        ).start()
        pltpu.make_async_copy(
            emb_hbm.at[pl.ds(base + a_rows, b_rows), :],
            emb_vmem.at[pl.ds(a_rows, b_rows), :],
            sem_b,
        ).start()
        pltpu.make_async_copy(
            emb_hbm.at[pl.ds(base + seq_len, shift), :], tail_vmem, sem_t
        ).start()
        pltpu.make_async_copy(
            emb_hbm.at[pl.ds(base, a_rows), :],
            emb_vmem.at[pl.ds(0, a_rows), :],
            sem_a,
        ).wait()

    _half(x_ref, o_ref, emb_vmem, 0, half, shift)

    @pl.when(b == 0)
    def _wait_rest():
        pltpu.make_async_copy(
            emb_hbm.at[pl.ds(base + a_rows, b_rows), :],
            emb_vmem.at[pl.ds(a_rows, b_rows), :],
            sem_b,
        ).wait()
        pltpu.make_async_copy(
            emb_hbm.at[pl.ds(base + seq_len, shift), :], tail_vmem, sem_t
        ).wait()
        pad = jnp.zeros((8 - shift, tail_vmem.shape[1]), tail_vmem.dtype)
        emb_vmem[pl.ds(seq_len, 8), :] = jnp.concatenate(
            [tail_vmem[...], pad], axis=0
        )

    _half(x_ref, o_ref, emb_vmem, half, half, shift)


def kernel(x, rel_pos_emb):
    batch, seq_len, d_model = x.shape
    base = (_MAX_LEN - 1) // 8 * 8
    shift = (_MAX_LEN - 1) - base
    body = functools.partial(_body, seq_len=seq_len, base=base, shift=shift)
    return pl.pallas_call(
        body,
        grid=(1, batch),
        in_specs=[
            pl.BlockSpec((1, seq_len, d_model), lambda i, b: (b, i, 0)),
            pl.BlockSpec(memory_space=pltpu.MemorySpace.HBM),
        ],
        out_specs=pl.BlockSpec((1, seq_len, d_model), lambda i, b: (b, i, 0)),
        out_shape=jax.ShapeDtypeStruct(x.shape, x.dtype),
        scratch_shapes=[
            pltpu.VMEM((seq_len + 8, d_model), x.dtype),
            pltpu.VMEM((shift, d_model), x.dtype),
            pltpu.SemaphoreType.DMA,
            pltpu.SemaphoreType.DMA,
            pltpu.SemaphoreType.DMA,
        ],
    )(x, rel_pos_emb)
